# baseline (speedup 1.0000x reference)
import functools

import jax
import jax.numpy as jnp
from jax import lax
from jax.experimental import pallas as pl
from jax.experimental.pallas import tpu as pltpu
from jax.experimental.pallas import tpu_sc as plsc

N = 10000
E = 320000
H = 128

NCORE = 2
NSUB = 16
CHUNK = 128
CHUNKS_PER_TILE = 80
EPT = CHUNKS_PER_TILE * CHUNK
EPC = EPT * NSUB
EPAD = EPC * NCORE
TRASH = N
NP = 10240
DEG_R = 10240
RPT = NP // NSUB

_mesh = plsc.VectorSubcoreMesh(core_axis_name="c", subcore_axis_name="s")


@functools.partial(
    pl.kernel,
    out_type=jax.ShapeDtypeStruct((NCORE, DEG_R), jnp.float32),
    mesh=_mesh,
    scratch_types=[
        pltpu.VMEM((CHUNK,), jnp.int32),
        pltpu.VMEM((CHUNK,), jnp.int32),
        pltpu.VMEM((CHUNK,), jnp.float32),
        pltpu.VMEM((640,), jnp.float32),
        pltpu.VMEM_SHARED((DEG_R,), jnp.float32),
    ],
)
def _deg_kernel(epk_hbm, deg_out, pk_v, col_v, ones_v, zeros_v, deg_sp):
    c = lax.axis_index("c")
    t = lax.axis_index("s")
    for j in range(CHUNK // 16):
        ones_v[pl.ds(j * 16, 16)] = jnp.ones((16,), jnp.float32)
    for j in range(640 // 16):
        zeros_v[pl.ds(j * 16, 16)] = jnp.zeros((16,), jnp.float32)
    pltpu.sync_copy(zeros_v, deg_sp.at[pl.ds(t * 640, 640)])
    plsc.subcore_barrier()

    ebase = c * EPC + t * EPT

    @pl.loop(0, CHUNKS_PER_TILE)
    def _(g):
        pltpu.sync_copy(epk_hbm.at[pl.ds(ebase + g * CHUNK, CHUNK)], pk_v)
        for j in range(CHUNK // 16):
            col_v[pl.ds(j * 16, 16)] = lax.shift_right_logical(
                pk_v[pl.ds(j * 16, 16)], jnp.int32(16))
        pltpu.sync_copy(ones_v, deg_sp.at[col_v], add=True)

    plsc.subcore_barrier()
    pltpu.sync_copy(deg_sp.at[pl.ds(t * 640, 640)],
                    deg_out.at[c, pl.ds(t * 640, 640)])


CH0 = 80
CH1 = 80
NCHT = CH0 + CH1
assert NCHT * NSUB * CHUNK == EPAD


def _make_agg_kernel(S):
    @functools.partial(
        pl.kernel,
        out_type=jax.ShapeDtypeStruct((NCORE, S, NP, H), jnp.float32),
        mesh=_mesh,
        scratch_types=(
            [pltpu.VMEM((CHUNK,), jnp.int32)] * 8
            + [pltpu.VMEM((CHUNK, H), jnp.float32)] * 2
            + [pltpu.VMEM_SHARED((NP, H), jnp.float32)]
            + [pltpu.SemaphoreType.DMA] * 10
        ),
    )
    def _agg(r0_hbm, r1_hbm, r2_hbm, r3_hbm, cols_hbm, z_hbm, out_hbm,
             g0, g1, g2, g3, c0_, c1_, c2_, c3_, b0, b1, acc_sp,
             ir0, ir1, ir2, ir3, ic0, ic1, ic2, ic3, gs0, gs1):
        gidx = (g0, g1, g2, g3)
        cidx = (c0_, c1_, c2_, c3_)
        buf = (b0, b1)
        isr = (ir0, ir1, ir2, ir3)
        isc = (ic0, ic1, ic2, ic3)
        gsem = (gs0, gs1)
        rows_hbm = (r0_hbm, r1_hbm, r2_hbm, r3_hbm)

        c = lax.axis_index("c")
        t = lax.axis_index("s")

        def run_slice(s, CH, cbase):
            rs_hbm = rows_hbm[s]

            @pl.loop(0, CH)
            def _(g):
                off = (cbase + g) * CHUNK
                pltpu.sync_copy(rs_hbm.at[pl.ds(off, CHUNK)], gidx[0])
                pltpu.sync_copy(cols_hbm.at[pl.ds(off, CHUNK)], cidx[0])
                pltpu.async_copy(z_hbm.at[gidx[0]], buf[0], gsem[0]).wait()
                pltpu.sync_copy(buf[0], acc_sp.at[cidx[0]], add=True)

        for s in range(S):
            pltpu.sync_copy(z_hbm.at[pl.ds(s * NP + t * RPT, RPT)],
                            acc_sp.at[pl.ds(t * RPT, RPT)])
            plsc.subcore_barrier()

            @pl.when(c == 0)
            def _():
                run_slice(s, CH0, t * CH0)

            @pl.when(c == 1)
            def _():
                run_slice(s, CH1, NSUB * CH0 + t * CH1)

            plsc.subcore_barrier()
            pltpu.sync_copy(acc_sp.at[pl.ds(t * RPT, RPT)],
                            out_hbm.at[c, s, pl.ds(t * RPT, RPT)])
            plsc.subcore_barrier()

    return _agg


_agg1 = _make_agg_kernel(1)
_agg2 = _make_agg_kernel(2)
_agg4 = _make_agg_kernel(4)


NB = 1000
GRID = N // NB


def _full(shape):
    return pl.BlockSpec(shape, lambda i: (0,) * len(shape))


MCH = EPAD // CHUNK
MB = MCH // GRID


def _prep_body(x_ref, wg_ref, d0_ref, d1_ref, rows_ref,
               z_ref, dinv_ref, r1_ref, r2_ref, r3_ref):
    dinv = lax.rsqrt(d0_ref[...] + d1_ref[...] + 1.0)
    xw = jnp.dot(x_ref[...], wg_ref[...], preferred_element_type=jnp.float32)
    z_ref[0] = dinv * xw
    dinv_ref[...] = dinv
    r = rows_ref[...]
    r1_ref[...] = r + jnp.int32(NP)
    r2_ref[...] = r + jnp.int32(2 * NP)
    r3_ref[...] = r + jnp.int32(3 * NP)


def _prep(x, Wg1, d0, d1, rows2d):
    return pl.pallas_call(
        _prep_body,
        grid=(GRID,),
        in_specs=[
            pl.BlockSpec((NB, H), lambda i: (i, 0)),
            _full((H, H)),
            pl.BlockSpec((NB, 1), lambda i: (i, 0)),
            pl.BlockSpec((NB, 1), lambda i: (i, 0)),
            pl.BlockSpec((MB, H), lambda i: (i, 0)),
        ],
        out_specs=[
            pl.BlockSpec((1, NB, H), lambda i: (0, i, 0)),
            pl.BlockSpec((NB, 1), lambda i: (i, 0)),
            pl.BlockSpec((MB, H), lambda i: (i, 0)),
            pl.BlockSpec((MB, H), lambda i: (i, 0)),
            pl.BlockSpec((MB, H), lambda i: (i, 0)),
        ],
        out_shape=[
            jax.ShapeDtypeStruct((1, NP, H), jnp.float32),
            jax.ShapeDtypeStruct((N, 1), jnp.float32),
            jax.ShapeDtypeStruct((MCH, H), jnp.int32),
            jax.ShapeDtypeStruct((MCH, H), jnp.int32),
            jax.ShapeDtypeStruct((MCH, H), jnp.int32),
        ],
    )(x, Wg1, d0, d1, rows2d)


def _encode_block(aggp, z, dinv, wf, b, S):
    h = b
    for s in range(S):
        u = dinv * (aggp[0, s] + aggp[1, s] - z[s])
        h = h + jnp.dot(u, wf[s * H:(s + 1) * H, :],
                        preferred_element_type=jnp.float32)
    return jax.nn.relu(h)


def _make_layer(S_in, S_out):
    def body(aggp_ref, z_ref, dinv_ref, wf_ref, b_ref, wg_ref, a_ref,
             znext_ref, part_ref):
        dinv = dinv_ref[...]
        xi = _encode_block(aggp_ref[...], z_ref[...], dinv, wf_ref[...],
                           b_ref[...], S_in)
        part_ref[...] = jnp.dot(xi, a_ref[...],
                                preferred_element_type=jnp.float32)
        xw = jnp.dot(xi, wg_ref[...], preferred_element_type=jnp.float32)
        for sp in range(S_out):
            znext_ref[sp] = dinv * xw[:, sp * H:(sp + 1) * H]

    Hi, Ho = S_in * H, S_out * H

    def run(aggp, z, dinv, wf, b, wg, a):
        return pl.pallas_call(
            body,
            grid=(GRID,),
            in_specs=[
                pl.BlockSpec((NCORE, S_in, NB, H), lambda i: (0, 0, i, 0)),
                pl.BlockSpec((S_in, NB, H), lambda i: (0, i, 0)),
                pl.BlockSpec((NB, 1), lambda i: (i, 0)),
                _full((Hi, Hi)),
                _full((1, Hi)),
                _full((Hi, Ho)),
                _full((Hi, H)),
            ],
            out_specs=[
                pl.BlockSpec((S_out, NB, H), lambda i: (0, i, 0)),
                pl.BlockSpec((NB, H), lambda i: (i, 0)),
            ],
            out_shape=[
                jax.ShapeDtypeStruct((S_out, NP, H), jnp.float32),
                jax.ShapeDtypeStruct((N, H), jnp.float32),
            ],
        )(aggp, z, dinv, wf, b, wg, a)

    return run


_layer1 = _make_layer(1, 2)
_layer2 = _make_layer(2, 4)


def _final_body(aggp_ref, z_ref, dinv_ref, wf_ref, b_ref, a_ref,
                p1_ref, p2_ref, c_ref, out_ref):
    xi = _encode_block(aggp_ref[...], z_ref[...], dinv_ref[...], wf_ref[...],
                       b_ref[...], 4)
    out_ref[...] = (jnp.dot(xi, a_ref[...], preferred_element_type=jnp.float32)
                    + p1_ref[...] + p2_ref[...] + c_ref[...])


def _final(aggp, z, dinv, wf, b, a, p1, p2, cvec):
    Hi = 4 * H
    return pl.pallas_call(
        _final_body,
        grid=(GRID,),
        in_specs=[
            pl.BlockSpec((NCORE, 4, NB, H), lambda i: (0, 0, i, 0)),
            pl.BlockSpec((4, NB, H), lambda i: (0, i, 0)),
            pl.BlockSpec((NB, 1), lambda i: (i, 0)),
            _full((Hi, Hi)),
            _full((1, Hi)),
            _full((Hi, H)),
            pl.BlockSpec((NB, H), lambda i: (i, 0)),
            pl.BlockSpec((NB, H), lambda i: (i, 0)),
            _full((1, H)),
        ],
        out_specs=pl.BlockSpec((NB, H), lambda i: (i, 0)),
        out_shape=jax.ShapeDtypeStruct((N, H), jnp.float32),
    )(aggp, z, dinv, wf, b, a, p1, p2, cvec)


def kernel(x, edge_index, Wg1, bg1, Wf1, bf1, Wg2, bg2, Wf2, bf2, Wg3, bg3,
           Wf3, bf3, Wd1, bd1, Wd2, bd2, Wd3, bd3, Ws1, bs1, Ws2, bs2,
           Wfin, bfin):
    Wfin3 = Wfin.reshape(H, 8, H)
    U = [Wfin3[:, p, :] for p in range(8)]
    B = [Wd3[:, :, 0] @ U[2 * m] + Wd3[:, :, 1] @ U[2 * m + 1]
         for m in range(4)]
    C = [Wd2[:, :, 0] @ B[2 * l] + Wd2[:, :, 1] @ B[2 * l + 1]
         for l in range(2)]
    A3 = Wd1[:, :, 0] @ C[0] + Wd1[:, :, 1] @ C[1]
    A2 = Ws1[:, :, 0].T @ (C[0] + C[1])
    A1 = Ws2[:, :, 0].T @ (B[0] + B[1] + B[2] + B[3])
    sumU = U[0] + U[1] + U[2] + U[3] + U[4] + U[5] + U[6] + U[7]
    sumB = B[0] + B[1] + B[2] + B[3]
    sumC = C[0] + C[1]
    cvec = (bd3 @ sumU + (bd2 + bs2) @ sumB + (bd1 + bs1) @ sumC
            + bfin).reshape(1, H)
    b1 = (bg1 @ Wf1 + bf1).reshape(1, H)
    b2 = (bg2 @ Wf2 + bf2).reshape(1, 2 * H)
    b3 = (bg3 @ Wf3 + bf3).reshape(1, 4 * H)

    rows = edge_index[0]
    cols = edge_index[1]
    rows_p = jnp.concatenate(
        [rows, jnp.zeros((EPAD - E,), jnp.int32)])
    cols_p = jnp.concatenate(
        [cols, jnp.full((EPAD - E,), TRASH, jnp.int32)])
    epk = jnp.bitwise_or(rows_p, jnp.left_shift(cols_p, 16))
    epk = jnp.concatenate([epk, jnp.zeros((CHUNK,), jnp.int32)])

    deg = _deg_kernel(epk)
    d0 = deg[0].reshape(DEG_R, 1)
    d1 = deg[1].reshape(DEG_R, 1)

    rows2d = rows_p.reshape(EPAD // CHUNK, CHUNK)
    z1, dinv, r1, r2, r3 = _prep(x, Wg1, d0, d1, rows2d)
    r1, r2, r3 = (r1.reshape(EPAD), r2.reshape(EPAD), r3.reshape(EPAD))
    aggp1 = _agg1(rows_p, r1, r2, r3, cols_p, z1.reshape(NP, H))
    z2, part1 = _layer1(aggp1, z1, dinv, Wf1, b1, Wg2, A1)
    aggp2 = _agg2(rows_p, r1, r2, r3, cols_p, z2.reshape(2 * NP, H))
    z3, part2 = _layer2(aggp2, z2, dinv, Wf2, b2, Wg3, A2)
    aggp3 = _agg4(rows_p, r1, r2, r3, cols_p, z3.reshape(4 * NP, H))
    return _final(aggp3, z3, dinv, Wf3, b3, A3, part1, part2, cvec)


# speedup vs baseline: 1.4495x; 1.4495x over previous
import functools

import jax
import jax.numpy as jnp
from jax import lax
from jax.experimental import pallas as pl
from jax.experimental.pallas import tpu as pltpu
from jax.experimental.pallas import tpu_sc as plsc

N = 10000
E = 320000
H = 128

NCORE = 2
NSUB = 16
CHUNK = 128
CHUNKS_PER_TILE = 79
EPT = CHUNKS_PER_TILE * CHUNK
EPC = EPT * NSUB
EPAD = EPC * NCORE
TRASH = N
NP = 10240
DEG_R = 10240
RPT = NP // NSUB

_mesh = plsc.VectorSubcoreMesh(core_axis_name="c", subcore_axis_name="s")


@functools.partial(
    pl.kernel,
    out_type=jax.ShapeDtypeStruct((NCORE, DEG_R), jnp.float32),
    mesh=_mesh,
    scratch_types=[
        pltpu.VMEM((CHUNK,), jnp.int32),
        pltpu.VMEM((CHUNK,), jnp.float32),
        pltpu.VMEM((640,), jnp.float32),
        pltpu.VMEM_SHARED((DEG_R,), jnp.float32),
    ],
)
def _deg_kernel(cols_hbm, deg_out, col_v, ones_v, zeros_v, deg_sp):
    c = lax.axis_index("c")
    t = lax.axis_index("s")
    for j in range(CHUNK // 16):
        ones_v[pl.ds(j * 16, 16)] = jnp.ones((16,), jnp.float32)
    for j in range(640 // 16):
        zeros_v[pl.ds(j * 16, 16)] = jnp.zeros((16,), jnp.float32)
    pltpu.sync_copy(zeros_v, deg_sp.at[pl.ds(t * 640, 640)])
    plsc.subcore_barrier()

    ebase = c * EPC + t * EPT

    @pl.loop(0, CHUNKS_PER_TILE)
    def _(g):
        pltpu.sync_copy(cols_hbm.at[pl.ds(ebase + g * CHUNK, CHUNK)], col_v)
        pltpu.sync_copy(ones_v, deg_sp.at[col_v], add=True)

    plsc.subcore_barrier()
    pltpu.sync_copy(deg_sp.at[pl.ds(t * 640, 640)],
                    deg_out.at[c, pl.ds(t * 640, 640)])


def _make_agg_kernel(S):
    @functools.partial(
        pl.kernel,
        out_type=jax.ShapeDtypeStruct((NCORE, S, NP, H), jnp.float32),
        mesh=_mesh,
        scratch_types=[
            pltpu.VMEM((CHUNK,), jnp.int32),
            pltpu.VMEM((CHUNK,), jnp.int32),
            pltpu.VMEM((CHUNK,), jnp.int32),
            pltpu.VMEM((CHUNK, H), jnp.float32),
            pltpu.VMEM_SHARED((NP, H), jnp.float32),
            pltpu.SemaphoreType.DMA,
        ],
    )
    def _agg(rows_hbm, cols_hbm, z_hbm, out_hbm,
             row_v, col_v, gidx_v, rows_v, acc_sp, sem):
        c = lax.axis_index("c")
        t = lax.axis_index("s")
        ebase = c * EPC + t * EPT

        for s in range(S):
            pltpu.sync_copy(z_hbm.at[pl.ds(s * NP + t * RPT, RPT)],
                            acc_sp.at[pl.ds(t * RPT, RPT)])
            plsc.subcore_barrier()

            @pl.loop(0, CHUNKS_PER_TILE)
            def _(g):
                off = ebase + g * CHUNK
                pltpu.sync_copy(rows_hbm.at[pl.ds(off, CHUNK)], row_v)
                pltpu.sync_copy(cols_hbm.at[pl.ds(off, CHUNK)], col_v)
                if s == 0:
                    gsrc = row_v
                else:
                    for j in range(CHUNK // 16):
                        gidx_v[pl.ds(j * 16, 16)] = (
                            row_v[pl.ds(j * 16, 16)]
                            + jnp.full((16,), s * NP, jnp.int32))
                    gsrc = gidx_v
                pltpu.async_copy(z_hbm.at[gsrc], rows_v, sem).wait()
                pltpu.sync_copy(rows_v, acc_sp.at[col_v], add=True)

            plsc.subcore_barrier()
            pltpu.sync_copy(acc_sp.at[pl.ds(t * RPT, RPT)],
                            out_hbm.at[c, s, pl.ds(t * RPT, RPT)])
            plsc.subcore_barrier()

    return _agg


_agg1 = _make_agg_kernel(1)
_agg2 = _make_agg_kernel(2)
_agg4 = _make_agg_kernel(4)


NB = 1000
GRID = N // NB


def _full(shape):
    return pl.BlockSpec(shape, lambda i: (0,) * len(shape))


def _prep_body(x_ref, wg_ref, d0_ref, d1_ref, z_ref, dinv_ref):
    dinv = lax.rsqrt(d0_ref[...] + d1_ref[...] + 1.0)
    xw = jnp.dot(x_ref[...], wg_ref[...], preferred_element_type=jnp.float32)
    z_ref[0] = dinv * xw
    dinv_ref[...] = dinv


def _prep(x, Wg1, d0, d1):
    return pl.pallas_call(
        _prep_body,
        grid=(GRID,),
        in_specs=[
            pl.BlockSpec((NB, H), lambda i: (i, 0)),
            _full((H, H)),
            pl.BlockSpec((NB, 1), lambda i: (i, 0)),
            pl.BlockSpec((NB, 1), lambda i: (i, 0)),
        ],
        out_specs=[
            pl.BlockSpec((1, NB, H), lambda i: (0, i, 0)),
            pl.BlockSpec((NB, 1), lambda i: (i, 0)),
        ],
        out_shape=[
            jax.ShapeDtypeStruct((1, NP, H), jnp.float32),
            jax.ShapeDtypeStruct((N, 1), jnp.float32),
        ],
    )(x, Wg1, d0, d1)


def _encode_block(aggp, z, dinv, wf, b, S):
    h = b
    for s in range(S):
        u = dinv * (aggp[0, s] + aggp[1, s] - z[s])
        h = h + jnp.dot(u, wf[s * H:(s + 1) * H, :],
                        preferred_element_type=jnp.float32)
    return jax.nn.relu(h)


def _make_layer(S_in, S_out):
    def body(aggp_ref, z_ref, dinv_ref, wf_ref, b_ref, wg_ref, a_ref,
             znext_ref, part_ref):
        dinv = dinv_ref[...]
        xi = _encode_block(aggp_ref[...], z_ref[...], dinv, wf_ref[...],
                           b_ref[...], S_in)
        part_ref[...] = jnp.dot(xi, a_ref[...],
                                preferred_element_type=jnp.float32)
        xw = jnp.dot(xi, wg_ref[...], preferred_element_type=jnp.float32)
        for sp in range(S_out):
            znext_ref[sp] = dinv * xw[:, sp * H:(sp + 1) * H]

    Hi, Ho = S_in * H, S_out * H

    def run(aggp, z, dinv, wf, b, wg, a):
        return pl.pallas_call(
            body,
            grid=(GRID,),
            in_specs=[
                pl.BlockSpec((NCORE, S_in, NB, H), lambda i: (0, 0, i, 0)),
                pl.BlockSpec((S_in, NB, H), lambda i: (0, i, 0)),
                pl.BlockSpec((NB, 1), lambda i: (i, 0)),
                _full((Hi, Hi)),
                _full((1, Hi)),
                _full((Hi, Ho)),
                _full((Hi, H)),
            ],
            out_specs=[
                pl.BlockSpec((S_out, NB, H), lambda i: (0, i, 0)),
                pl.BlockSpec((NB, H), lambda i: (i, 0)),
            ],
            out_shape=[
                jax.ShapeDtypeStruct((S_out, NP, H), jnp.float32),
                jax.ShapeDtypeStruct((N, H), jnp.float32),
            ],
        )(aggp, z, dinv, wf, b, wg, a)

    return run


_layer1 = _make_layer(1, 2)
_layer2 = _make_layer(2, 4)


def _final_body(aggp_ref, z_ref, dinv_ref, wf_ref, b_ref, a_ref,
                p1_ref, p2_ref, c_ref, out_ref):
    xi = _encode_block(aggp_ref[...], z_ref[...], dinv_ref[...], wf_ref[...],
                       b_ref[...], 4)
    out_ref[...] = (jnp.dot(xi, a_ref[...], preferred_element_type=jnp.float32)
                    + p1_ref[...] + p2_ref[...] + c_ref[...])


def _final(aggp, z, dinv, wf, b, a, p1, p2, cvec):
    Hi = 4 * H
    return pl.pallas_call(
        _final_body,
        grid=(GRID,),
        in_specs=[
            pl.BlockSpec((NCORE, 4, NB, H), lambda i: (0, 0, i, 0)),
            pl.BlockSpec((4, NB, H), lambda i: (0, i, 0)),
            pl.BlockSpec((NB, 1), lambda i: (i, 0)),
            _full((Hi, Hi)),
            _full((1, Hi)),
            _full((Hi, H)),
            pl.BlockSpec((NB, H), lambda i: (i, 0)),
            pl.BlockSpec((NB, H), lambda i: (i, 0)),
            _full((1, H)),
        ],
        out_specs=pl.BlockSpec((NB, H), lambda i: (i, 0)),
        out_shape=jax.ShapeDtypeStruct((N, H), jnp.float32),
    )(aggp, z, dinv, wf, b, a, p1, p2, cvec)


def kernel(x, edge_index, Wg1, bg1, Wf1, bf1, Wg2, bg2, Wf2, bf2, Wg3, bg3,
           Wf3, bf3, Wd1, bd1, Wd2, bd2, Wd3, bd3, Ws1, bs1, Ws2, bs2,
           Wfin, bfin):
    Wfin3 = Wfin.reshape(H, 8, H)
    U = [Wfin3[:, p, :] for p in range(8)]
    B = [Wd3[:, :, 0] @ U[2 * m] + Wd3[:, :, 1] @ U[2 * m + 1]
         for m in range(4)]
    C = [Wd2[:, :, 0] @ B[2 * l] + Wd2[:, :, 1] @ B[2 * l + 1]
         for l in range(2)]
    A3 = Wd1[:, :, 0] @ C[0] + Wd1[:, :, 1] @ C[1]
    A2 = Ws1[:, :, 0].T @ (C[0] + C[1])
    A1 = Ws2[:, :, 0].T @ (B[0] + B[1] + B[2] + B[3])
    sumU = U[0] + U[1] + U[2] + U[3] + U[4] + U[5] + U[6] + U[7]
    sumB = B[0] + B[1] + B[2] + B[3]
    sumC = C[0] + C[1]
    cvec = (bd3 @ sumU + (bd2 + bs2) @ sumB + (bd1 + bs1) @ sumC
            + bfin).reshape(1, H)
    b1 = (bg1 @ Wf1 + bf1).reshape(1, H)
    b2 = (bg2 @ Wf2 + bf2).reshape(1, 2 * H)
    b3 = (bg3 @ Wf3 + bf3).reshape(1, 4 * H)

    rows = edge_index[0]
    cols = edge_index[1]
    rows_p = jnp.concatenate(
        [rows, jnp.zeros((EPAD - E,), jnp.int32)])
    cols_p = jnp.concatenate(
        [cols, jnp.full((EPAD - E,), TRASH, jnp.int32)])

    deg = _deg_kernel(cols_p)
    d0 = deg[0].reshape(DEG_R, 1)
    d1 = deg[1].reshape(DEG_R, 1)

    z1, dinv = _prep(x, Wg1, d0, d1)
    aggp1 = _agg1(rows_p, cols_p, z1.reshape(NP, H))
    z2, part1 = _layer1(aggp1, z1, dinv, Wf1, b1, Wg2, A1)
    aggp2 = _agg2(rows_p, cols_p, z2.reshape(2 * NP, H))
    z3, part2 = _layer2(aggp2, z2, dinv, Wf2, b2, Wg3, A2)
    aggp3 = _agg4(rows_p, cols_p, z3.reshape(4 * NP, H))
    return _final(aggp3, z3, dinv, Wf3, b3, A3, part1, part2, cvec)
